# baseline (speedup 1.0000x reference)
import jax
import jax.numpy as jnp
from jax import lax
from jax.experimental import pallas as pl
from jax.experimental.pallas import tpu as pltpu
from jax.experimental.pallas import tpu_sc as plsc

NUM_CORES = 2
NUM_SUBCORES = 16
NUM_WORKERS = NUM_CORES * NUM_SUBCORES
BATCH = 16384
NROWS = 1000000
EMBED_DIM = 32
LANES = 16
B_PER_W = BATCH // NUM_WORKERS
CHUNK = 128
N_CHUNKS = B_PER_W // CHUNK
GROUPS = CHUNK // LANES


def _mf_body(uid_hbm, iid_hbm, ut_hbm, it_hbm, ub_hbm, ib_hbm, gb_hbm,
             out_hbm,
             uids_v, iids_v, urows_v, irows_v, ub_v, ib_v, gb_v, out_v,
             bsem, gsem):
    wid = lax.axis_index("s") * NUM_CORES + lax.axis_index("c")
    base = wid * B_PER_W

    pltpu.sync_copy(uid_hbm.at[pl.ds(wid * N_CHUNKS, N_CHUNKS)], uids_v)
    pltpu.sync_copy(iid_hbm.at[pl.ds(wid * N_CHUNKS, N_CHUNKS)], iids_v)
    pltpu.sync_copy(gb_hbm, gb_v)

    bias_copies = []
    for q in range(N_CHUNKS):
        sl = pl.ds(q * CHUNK, CHUNK)
        bias_copies.append(
            pltpu.async_copy(ub_hbm.at[uids_v.at[q]], ub_v.at[sl], bsem))
        bias_copies.append(
            pltpu.async_copy(ib_hbm.at[iids_v.at[q]], ib_v.at[sl], bsem))

    def fire(c):
        buf = c % 2
        sem = gsem.at[buf]
        for d in range(EMBED_DIM):
            pltpu.async_copy(ut_hbm.at[d].at[uids_v.at[c]],
                             urows_v.at[buf, d], sem)
            pltpu.async_copy(it_hbm.at[d].at[iids_v.at[c]],
                             irows_v.at[buf, d], sem)

    def drain(c):
        buf = c % 2
        sem = gsem.at[buf]
        for d in range(EMBED_DIM):
            pltpu.make_async_copy(ut_hbm.at[d].at[uids_v.at[c]],
                                  urows_v.at[buf, d], sem).wait()
            pltpu.make_async_copy(it_hbm.at[d].at[iids_v.at[c]],
                                  irows_v.at[buf, d], sem).wait()

    fire(0)

    for c in range(N_CHUNKS):
        if c + 1 < N_CHUNKS:
            fire(c + 1)
        drain(c)
        if c == 0:
            for cp in bias_copies:
                cp.wait()
        buf = c % 2
        for g in range(GROUPS):
            ssl = pl.ds(g * LANES, LANES)
            acc = gb_v[...]
            for d in range(EMBED_DIM):
                acc = acc + urows_v[buf, d, ssl] * irows_v[buf, d, ssl]
            osl = pl.ds(c * CHUNK + g * LANES, LANES)
            out_v[osl] = acc + ub_v[osl] + ib_v[osl]

    pltpu.sync_copy(out_v, out_hbm.at[pl.ds(base, B_PER_W)])


@jax.jit
def kernel(user_ids, item_ids, user_table, item_table, user_bias, item_bias,
           global_bias):
    uid2 = user_ids.astype(jnp.int32).reshape(BATCH // CHUNK, CHUNK)
    iid2 = item_ids.astype(jnp.int32).reshape(BATCH // CHUNK, CHUNK)
    ut_t = user_table.T
    it_t = item_table.T
    ub_flat = user_bias.reshape(-1)
    ib_flat = item_bias.reshape(-1)
    gb = jnp.broadcast_to(global_bias.reshape(1), (LANES,))

    mesh = plsc.VectorSubcoreMesh(
        core_axis_name="c", subcore_axis_name="s",
        num_cores=NUM_CORES, num_subcores=NUM_SUBCORES)

    run = pl.kernel(
        _mf_body,
        out_type=jax.ShapeDtypeStruct((BATCH,), jnp.float32),
        mesh=mesh,
        compiler_params=pltpu.CompilerParams(
            needs_layout_passes=False, use_tc_tiling_on_sc=False),
        scratch_types=[
            pltpu.VMEM((N_CHUNKS, CHUNK), jnp.int32),
            pltpu.VMEM((N_CHUNKS, CHUNK), jnp.int32),
            pltpu.VMEM((2, EMBED_DIM, CHUNK), jnp.float32),
            pltpu.VMEM((2, EMBED_DIM, CHUNK), jnp.float32),
            pltpu.VMEM((B_PER_W,), jnp.float32),
            pltpu.VMEM((B_PER_W,), jnp.float32),
            pltpu.VMEM((LANES,), jnp.float32),
            pltpu.VMEM((B_PER_W,), jnp.float32),
            pltpu.SemaphoreType.DMA,
            pltpu.SemaphoreType.DMA((2,)),
        ],
    )
    return run(uid2, iid2, ut_t, it_t, ub_flat, ib_flat, gb)


# speedup vs baseline: 5.8145x; 5.8145x over previous
import jax
import jax.numpy as jnp
from jax import lax
from jax.experimental import pallas as pl
from jax.experimental.pallas import tpu as pltpu
from jax.experimental.pallas import tpu_sc as plsc

NUM_CORES = 2
NUM_SUBCORES = 16
NUM_WORKERS = NUM_CORES * NUM_SUBCORES
BATCH = 16384
EMBED_DIM = 32
LANES = 16
B_PER_W = BATCH // NUM_WORKERS
CHUNK = 128
N_CHUNKS = B_PER_W // CHUNK


def _mf_body(uid_hbm, iid_hbm, ut_hbm, it_hbm, ub_hbm, ib_hbm, gb_hbm,
             out_hbm,
             uidx_v, iidx_v, urows_v, irows_v, ub_v, ib_v, gb_v, out_v, sem):
    wid = lax.axis_index("s") * NUM_CORES + lax.axis_index("c")
    base = wid * B_PER_W
    idx_row_base = wid * N_CHUNKS

    pltpu.sync_copy(uid_hbm.at[pl.ds(idx_row_base, N_CHUNKS)], uidx_v)
    pltpu.sync_copy(iid_hbm.at[pl.ds(idx_row_base, N_CHUNKS)], iidx_v)
    pltpu.sync_copy(gb_hbm, gb_v)

    copies = []
    for c in range(N_CHUNKS):
        sl = pl.ds(c * CHUNK, CHUNK)
        copies.append(
            pltpu.async_copy(ut_hbm.at[uidx_v.at[c]], urows_v.at[sl], sem))
        copies.append(
            pltpu.async_copy(it_hbm.at[iidx_v.at[c]], irows_v.at[sl], sem))
        copies.append(
            pltpu.async_copy(ub_hbm.at[uidx_v.at[c]], ub_v.at[sl], sem))
        copies.append(
            pltpu.async_copy(ib_hbm.at[iidx_v.at[c]], ib_v.at[sl], sem))
    for cp in copies:
        cp.wait()

    gb_vec = gb_v[...]
    lane = lax.iota(jnp.int32, LANES)

    def group_body(g, carry):
        r0 = g * LANES
        acc = jnp.zeros((LANES,), jnp.float32)
        for j in range(LANES):
            r = r0 + j
            u0 = urows_v[r, pl.ds(0, LANES)]
            u1 = urows_v[r, pl.ds(LANES, LANES)]
            i0 = irows_v[r, pl.ds(0, LANES)]
            i1 = irows_v[r, pl.ds(LANES, LANES)]
            dot = jnp.sum(u0 * i0 + u1 * i1)
            acc = jnp.where(lane == j, dot, acc)
        sl = pl.ds(r0, LANES)
        out_v[sl] = acc + ub_v[sl] + ib_v[sl] + gb_vec
        return carry

    lax.fori_loop(0, B_PER_W // LANES, group_body, 0)

    pltpu.sync_copy(out_v, out_hbm.at[pl.ds(base, B_PER_W)])


@jax.jit
def kernel(user_ids, item_ids, user_table, item_table, user_bias, item_bias,
           global_bias):
    uid2 = user_ids.astype(jnp.int32).reshape(BATCH // CHUNK, CHUNK)
    iid2 = item_ids.astype(jnp.int32).reshape(BATCH // CHUNK, CHUNK)
    ub_flat = user_bias.reshape(-1)
    ib_flat = item_bias.reshape(-1)
    gb = jnp.broadcast_to(global_bias.reshape(1), (LANES,))

    mesh = plsc.VectorSubcoreMesh(
        core_axis_name="c", subcore_axis_name="s",
        num_cores=NUM_CORES, num_subcores=NUM_SUBCORES)

    run = pl.kernel(
        _mf_body,
        out_type=jax.ShapeDtypeStruct((BATCH,), jnp.float32),
        mesh=mesh,
        compiler_params=pltpu.CompilerParams(
            needs_layout_passes=False, use_tc_tiling_on_sc=False),
        scratch_types=[
            pltpu.VMEM((N_CHUNKS, CHUNK), jnp.int32),
            pltpu.VMEM((N_CHUNKS, CHUNK), jnp.int32),
            pltpu.VMEM((B_PER_W, EMBED_DIM), jnp.float32),
            pltpu.VMEM((B_PER_W, EMBED_DIM), jnp.float32),
            pltpu.VMEM((B_PER_W,), jnp.float32),
            pltpu.VMEM((B_PER_W,), jnp.float32),
            pltpu.VMEM((LANES,), jnp.float32),
            pltpu.VMEM((B_PER_W,), jnp.float32),
            pltpu.SemaphoreType.DMA,
        ],
    )
    return run(uid2, iid2, user_table, item_table, ub_flat, ib_flat, gb)
